# Initial kernel scaffold; baseline (speedup 1.0000x reference)
import jax
import jax.numpy as jnp
from jax.experimental import pallas as pl


def kernel(x, edge_index, W1, b1, W2, b2):
    raise NotImplementedError("write your pallas kernel here")


# speedup vs baseline: 26.0125x; 26.0125x over previous
import functools

import jax
import jax.numpy as jnp
from jax import lax
from jax.experimental import pallas as pl
from jax.experimental.pallas import tpu as pltpu
from jax.experimental.pallas import tpu_sc as plsc

N = 10000
E = 320000
C = 32
K = 10
ALPHA = 0.1

NP_ = 10240
CH = NP_ // 16
NTILES = 32
EPT = E // NTILES
KB = 128
EB = (EPT + KB - 1) // KB
EBP = 80
PAD_ROWS = NP_ - N


def _sc_body(g_hbm, src_hbm, dst_hbm, zeros_hbm, out_hbm,
             src_v, dst_v, rows_v, g_sh, agg_sh, sem_g):
    c = lax.axis_index("c")
    s = lax.axis_index("s")
    w = c * 16 + s

    pltpu.sync_copy(src_hbm.at[w], src_v)
    pltpu.sync_copy(dst_hbm.at[w], dst_v)
    pltpu.sync_copy(g_hbm.at[pl.ds(s * CH, CH)], g_sh.at[pl.ds(s * CH, CH)])
    pltpu.sync_copy(zeros_hbm, agg_sh.at[pl.ds(s * CH, CH)])
    plsc.subcore_barrier()

    def body(j, carry):
        pltpu.async_copy(g_sh.at[src_v.at[j]], rows_v, sem_g).wait()
        pltpu.sync_copy(rows_v, agg_sh.at[dst_v.at[j]], add=True)
        return carry

    lax.fori_loop(0, EBP, body, 0)
    plsc.subcore_barrier()
    pltpu.sync_copy(agg_sh.at[pl.ds(s * CH, CH)],
                    out_hbm.at[c, pl.ds(s * CH, CH)])


_sc_scatter = functools.partial(
    pl.kernel,
    out_type=jax.ShapeDtypeStruct((2, NP_, C), jnp.float32),
    mesh=plsc.VectorSubcoreMesh(core_axis_name="c", subcore_axis_name="s"),
    compiler_params=pltpu.CompilerParams(use_tc_tiling_on_sc=False),
    scratch_types=[
        pltpu.VMEM((EBP, KB), jnp.int32),
        pltpu.VMEM((EBP, KB), jnp.int32),
        pltpu.VMEM((KB, C), jnp.float32),
        pltpu.VMEM_SHARED((NP_, C), jnp.float32),
        pltpu.VMEM_SHARED((NP_, C), jnp.float32),
        pltpu.SemaphoreType.DMA,
    ],
)(_sc_body)


def _mlp_body(x_ref, w1_ref, b1_ref, w2_ref, b2_ref, z_ref):
    h = jnp.maximum(
        jax.lax.dot_general(x_ref[...], w1_ref[...], (((1,), (0,)), ((), ())),
                            preferred_element_type=jnp.float32) + b1_ref[...],
        0.0)
    z_ref[...] = jax.lax.dot_general(h, w2_ref[...], (((1,), (0,)), ((), ())),
                                     preferred_element_type=jnp.float32) + b2_ref[...]


def _mlp(x, W1, b1, W2, b2):
    nblk = 10
    rows = N // nblk
    return pl.pallas_call(
        _mlp_body,
        grid=(nblk,),
        in_specs=[
            pl.BlockSpec((rows, 128), lambda i: (i, 0)),
            pl.BlockSpec((128, 256), lambda i: (0, 0)),
            pl.BlockSpec((1, 256), lambda i: (0, 0)),
            pl.BlockSpec((256, C), lambda i: (0, 0)),
            pl.BlockSpec((1, C), lambda i: (0, 0)),
        ],
        out_specs=pl.BlockSpec((rows, C), lambda i: (i, 0)),
        out_shape=jax.ShapeDtypeStruct((N, C), jnp.float32),
    )(x, W1, b1.reshape(1, 256), W2, b2.reshape(1, C))


def _prep_body(da_ref, db_ref, zp_ref, d2_ref, zz_ref, g0_ref, sq_ref):
    deg = da_ref[...] + db_ref[...] + 1.0
    dis = jax.lax.rsqrt(deg)
    zp = zp_ref[...]
    d2_ref[...] = 1.0 / deg
    zz_ref[...] = ALPHA * dis * zp
    g0_ref[...] = dis * zp
    sq_ref[...] = jnp.sqrt(deg)


def _prep(deg2, zp):
    shp = jax.ShapeDtypeStruct((NP_, C), jnp.float32)
    return pl.pallas_call(
        _prep_body,
        out_shape=(shp, shp, shp, shp),
    )(deg2[0], deg2[1], zp)


def _update_body(aa_ref, ab_ref, g_ref, d2_ref, zz_ref, o_ref):
    o_ref[...] = (d2_ref[...] * ((1.0 - ALPHA)
                                 * (aa_ref[...] + ab_ref[...] + g_ref[...]))
                  + zz_ref[...])


def _update(agg2, g, d2f, zzf):
    return pl.pallas_call(
        _update_body,
        out_shape=jax.ShapeDtypeStruct((NP_, C), jnp.float32),
    )(agg2[0], agg2[1], g, d2f, zzf)


def _final_body(g_ref, sq_ref, lp_ref, h_ref):
    h = g_ref[...] * sq_ref[...]
    m = jnp.max(h, axis=1, keepdims=True)
    e = jnp.exp(h - m)
    ssum = jnp.sum(e, axis=1, keepdims=True)
    lp_ref[...] = (h - m) - jnp.log(ssum)
    h_ref[...] = h


def _final(g, sqf):
    shp = jax.ShapeDtypeStruct((N, C), jnp.float32)
    return pl.pallas_call(
        _final_body,
        out_shape=(shp, shp),
    )(g[:N], sqf[:N])


def kernel(x, edge_index, W1, b1, W2, b2):
    src = edge_index[0].reshape(NTILES, EPT)
    dst = edge_index[1].reshape(NTILES, EPT)
    npad = EBP * KB - EPT
    padidx = N + (jnp.arange(npad, dtype=jnp.int32) % PAD_ROWS)
    padblk = jnp.broadcast_to(padidx, (NTILES, npad))
    src_p = jnp.concatenate([src, padblk], axis=1).reshape(NTILES, EBP, KB)
    dst_p = jnp.concatenate([dst, padblk], axis=1).reshape(NTILES, EBP, KB)

    zeros_blk = jnp.zeros((CH, C), dtype=jnp.float32)
    ones_g = jnp.ones((NP_, C), dtype=jnp.float32)

    z = _mlp(x, W1, b1, W2, b2)
    zp = jnp.pad(z, ((0, PAD_ROWS), (0, 0)))

    deg2 = _sc_scatter(ones_g, src_p, dst_p, zeros_blk)
    d2f, zzf, g, sqf = _prep(deg2, zp)

    for _ in range(K):
        agg2 = _sc_scatter(g, src_p, dst_p, zeros_blk)
        g = _update(agg2, g, d2f, zzf)

    return _final(g, sqf)


# speedup vs baseline: 30.1733x; 1.1600x over previous
import functools

import jax
import jax.numpy as jnp
from jax import lax
from jax.experimental import pallas as pl
from jax.experimental.pallas import tpu as pltpu
from jax.experimental.pallas import tpu_sc as plsc

N = 10000
E = 320000
C = 32
K = 10
ALPHA = 0.1

NP_ = 10240
CH = NP_ // 16
NTILES = 32
EPT = E // NTILES
KB = 128
EB = (EPT + KB - 1) // KB
EBP = 80
PAD_ROWS = NP_ - N


def _sc_body(g_hbm, src_hbm, dst_hbm, zeros_hbm, out_hbm,
             src_v, dst_v, rows0, rows1, g_sh, agg_sh, sem0, sem1):
    c = lax.axis_index("c")
    s = lax.axis_index("s")
    w = c * 16 + s

    pltpu.sync_copy(src_hbm.at[w], src_v)
    pltpu.sync_copy(dst_hbm.at[w], dst_v)
    pltpu.sync_copy(g_hbm.at[pl.ds(s * CH, CH)], g_sh.at[pl.ds(s * CH, CH)])
    pltpu.sync_copy(zeros_hbm, agg_sh.at[pl.ds(s * CH, CH)])
    plsc.subcore_barrier()

    pltpu.async_copy(g_sh.at[src_v.at[0]], rows0, sem0)

    def body(jj, carry):
        j = 2 * jj
        pltpu.make_async_copy(g_sh.at[src_v.at[j]], rows0, sem0).wait()
        pltpu.async_copy(g_sh.at[src_v.at[j + 1]], rows1, sem1)
        pltpu.sync_copy(rows0, agg_sh.at[dst_v.at[j]], add=True)
        pltpu.make_async_copy(g_sh.at[src_v.at[j + 1]], rows1, sem1).wait()

        @pl.when(jj < EBP // 2 - 1)
        def _():
            pltpu.async_copy(g_sh.at[src_v.at[j + 2]], rows0, sem0)

        pltpu.sync_copy(rows1, agg_sh.at[dst_v.at[j + 1]], add=True)
        return carry

    lax.fori_loop(0, EBP // 2, body, 0)
    plsc.subcore_barrier()
    pltpu.sync_copy(agg_sh.at[pl.ds(s * CH, CH)],
                    out_hbm.at[c, pl.ds(s * CH, CH)])


_sc_scatter = functools.partial(
    pl.kernel,
    out_type=jax.ShapeDtypeStruct((2, NP_, C), jnp.float32),
    mesh=plsc.VectorSubcoreMesh(core_axis_name="c", subcore_axis_name="s"),
    compiler_params=pltpu.CompilerParams(use_tc_tiling_on_sc=False),
    scratch_types=[
        pltpu.VMEM((EBP, KB), jnp.int32),
        pltpu.VMEM((EBP, KB), jnp.int32),
        pltpu.VMEM((KB, C), jnp.float32),
        pltpu.VMEM((KB, C), jnp.float32),
        pltpu.VMEM_SHARED((NP_, C), jnp.float32),
        pltpu.VMEM_SHARED((NP_, C), jnp.float32),
        pltpu.SemaphoreType.DMA,
        pltpu.SemaphoreType.DMA,
    ],
)(_sc_body)


def _mlp_body(x_ref, w1_ref, b1_ref, w2_ref, b2_ref, z_ref):
    h = jnp.maximum(
        jax.lax.dot_general(x_ref[...], w1_ref[...], (((1,), (0,)), ((), ())),
                            preferred_element_type=jnp.float32) + b1_ref[...],
        0.0)
    z_ref[...] = jax.lax.dot_general(h, w2_ref[...], (((1,), (0,)), ((), ())),
                                     preferred_element_type=jnp.float32) + b2_ref[...]


def _mlp(x, W1, b1, W2, b2):
    nblk = 10
    rows = N // nblk
    return pl.pallas_call(
        _mlp_body,
        grid=(nblk,),
        in_specs=[
            pl.BlockSpec((rows, 128), lambda i: (i, 0)),
            pl.BlockSpec((128, 256), lambda i: (0, 0)),
            pl.BlockSpec((1, 256), lambda i: (0, 0)),
            pl.BlockSpec((256, C), lambda i: (0, 0)),
            pl.BlockSpec((1, C), lambda i: (0, 0)),
        ],
        out_specs=pl.BlockSpec((rows, C), lambda i: (i, 0)),
        out_shape=jax.ShapeDtypeStruct((N, C), jnp.float32),
    )(x, W1, b1.reshape(1, 256), W2, b2.reshape(1, C))


def _prep_body(da_ref, db_ref, zp_ref, d2_ref, zz_ref, g0_ref, sq_ref):
    deg = da_ref[...] + db_ref[...] + 1.0
    dis = jax.lax.rsqrt(deg)
    zp = zp_ref[...]
    d2_ref[...] = 1.0 / deg
    zz_ref[...] = ALPHA * dis * zp
    g0_ref[...] = dis * zp
    sq_ref[...] = jnp.sqrt(deg)


def _prep(deg2, zp):
    shp = jax.ShapeDtypeStruct((NP_, C), jnp.float32)
    return pl.pallas_call(
        _prep_body,
        out_shape=(shp, shp, shp, shp),
    )(deg2[0], deg2[1], zp)


def _update_body(aa_ref, ab_ref, g_ref, d2_ref, zz_ref, o_ref):
    o_ref[...] = (d2_ref[...] * ((1.0 - ALPHA)
                                 * (aa_ref[...] + ab_ref[...] + g_ref[...]))
                  + zz_ref[...])


def _update(agg2, g, d2f, zzf):
    return pl.pallas_call(
        _update_body,
        out_shape=jax.ShapeDtypeStruct((NP_, C), jnp.float32),
    )(agg2[0], agg2[1], g, d2f, zzf)


def _final_body(g_ref, sq_ref, lp_ref, h_ref):
    h = g_ref[...] * sq_ref[...]
    m = jnp.max(h, axis=1, keepdims=True)
    e = jnp.exp(h - m)
    ssum = jnp.sum(e, axis=1, keepdims=True)
    lp_ref[...] = (h - m) - jnp.log(ssum)
    h_ref[...] = h


def _final(g, sqf):
    shp = jax.ShapeDtypeStruct((N, C), jnp.float32)
    return pl.pallas_call(
        _final_body,
        out_shape=(shp, shp),
    )(g[:N], sqf[:N])


def kernel(x, edge_index, W1, b1, W2, b2):
    src = edge_index[0].reshape(NTILES, EPT)
    dst = edge_index[1].reshape(NTILES, EPT)
    npad = EBP * KB - EPT
    padidx = N + (jnp.arange(npad, dtype=jnp.int32) % PAD_ROWS)
    padblk = jnp.broadcast_to(padidx, (NTILES, npad))
    src_p = jnp.concatenate([src, padblk], axis=1).reshape(NTILES, EBP, KB)
    dst_p = jnp.concatenate([dst, padblk], axis=1).reshape(NTILES, EBP, KB)

    zeros_blk = jnp.zeros((CH, C), dtype=jnp.float32)
    ones_g = jnp.ones((NP_, C), dtype=jnp.float32)

    z = _mlp(x, W1, b1, W2, b2)
    zp = jnp.pad(z, ((0, PAD_ROWS), (0, 0)))

    deg2 = _sc_scatter(ones_g, src_p, dst_p, zeros_blk)
    d2f, zzf, g, sqf = _prep(deg2, zp)

    for _ in range(K):
        agg2 = _sc_scatter(g, src_p, dst_p, zeros_blk)
        g = _update(agg2, g, d2f, zzf)

    return _final(g, sqf)
